# baseline (speedup 1.0000x reference)
import functools

import jax
import jax.numpy as jnp
from jax.experimental import pallas as pl
from jax.experimental.pallas import tpu as pltpu

_ALPHA, _GAMMA = 0.25, 2.0
_C_CLASS, _C_BBOX, _C_GIOU = 2.0, 5.0, 2.0


def _cost_kernel(logits_ref, pred_ref, labels_ref, tbox_ref, out_ref, *, num_classes):
    nt = labels_ref.shape[1]
    q = logits_ref.shape[1]

    p = jax.nn.sigmoid(logits_ref[0])
    neg_cost = (1.0 - _ALPHA) * (p * p) * -jnp.log(1.0 - p + 1e-8)
    one_m_p = 1.0 - p
    pos_cost = _ALPHA * (one_m_p * one_m_p) * -jnp.log(p + 1e-8)
    cc = _C_CLASS * (pos_cost - neg_cost)

    pred = pred_ref[0]
    pcx, pcy = pred[:, 0:1], pred[:, 1:2]
    pw, ph = pred[:, 2:3], pred[:, 3:4]
    tcx, tcy = tbox_ref[0:1, :], tbox_ref[1:2, :]
    tw, th = tbox_ref[2:3, :], tbox_ref[3:4, :]

    rowsum = _C_BBOX * (((pcx + pcy) + (pw + ph)))
    colsum = _C_BBOX * (((tcx + tcy) + (tw + th))) + _C_GIOU
    lhs = jnp.concatenate(
        [cc, rowsum, jnp.ones((q, 1), jnp.float32)], axis=1)
    onehot = (jax.lax.broadcasted_iota(jnp.int32, (num_classes, nt), 0)
              == labels_ref[...]).astype(jnp.float32)
    rhs = jnp.concatenate(
        [onehot, jnp.ones((1, nt), jnp.float32), colsum], axis=0)
    m = jnp.dot(lhs, rhs, preferred_element_type=jnp.float32)

    t5 = (jnp.minimum(_C_BBOX * pcx, _C_BBOX * tcx)
          + jnp.minimum(_C_BBOX * pcy, _C_BBOX * tcy)
          + jnp.minimum(_C_BBOX * pw, _C_BBOX * tw)
          + jnp.minimum(_C_BBOX * ph, _C_BBOX * th))

    px0, px1 = pcx - 0.5 * pw, pcx + 0.5 * pw
    py0, py1 = pcy - 0.5 * ph, pcy + 0.5 * ph
    tx0, tx1 = tcx - 0.5 * tw, tcx + 0.5 * tw
    ty0, ty1 = tcy - 0.5 * th, tcy + 0.5 * th
    area_p = (px1 - px0) * (py1 - py0)
    area_t = (tx1 - tx0) * (ty1 - ty0)
    iw_raw = jnp.minimum(px1, tx1) - jnp.maximum(px0, tx0)
    ih_raw = jnp.minimum(py1, ty1) - jnp.maximum(py0, ty0)
    inter = jnp.maximum(iw_raw, 0.0) * jnp.maximum(ih_raw, 0.0)
    union = (area_p + area_t) - inter
    enclose = ((pw + tw) - iw_raw) * ((ph + th) - ih_raw)
    g = inter * (1.0 / union) + union * (1.0 / enclose)
    t = t5 + g
    out_ref[0] = m - (t + t)


def kernel(logits, pred_boxes, target_labels, target_boxes):
    batch, num_queries, num_classes = logits.shape
    nt = target_labels.shape[0]

    labels2 = target_labels.astype(jnp.int32).reshape(1, nt)
    tbox_t = target_boxes.T

    grid = (batch,)

    return pl.pallas_call(
        functools.partial(_cost_kernel, num_classes=num_classes),
        grid=grid,
        in_specs=[
            pl.BlockSpec((1, num_queries, num_classes), lambda i: (i, 0, 0)),
            pl.BlockSpec((1, num_queries, 4), lambda i: (i, 0, 0)),
            pl.BlockSpec((1, nt), lambda i: (0, 0)),
            pl.BlockSpec((4, nt), lambda i: (0, 0)),
        ],
        out_specs=pl.BlockSpec((1, num_queries, nt), lambda i: (i, 0, 0)),
        out_shape=jax.ShapeDtypeStruct((batch, num_queries, nt), jnp.float32),
        compiler_params=pltpu.CompilerParams(
            dimension_semantics=("parallel",),
        ),
    )(logits, pred_boxes, labels2, tbox_t)
